# baseline (speedup 1.0000x reference)
import functools

import jax
import jax.numpy as jnp
from jax import lax
from jax.experimental import pallas as pl
from jax.experimental.pallas import tpu as pltpu
from jax.experimental.pallas import tpu_sc as plsc

B = 4096
T = 200
D = 64
VOCAB = 1000000
SCALE = 8.0

_NC = 2
_NS = 16
_NW = _NC * _NS
_BW = B // _NW

_mesh = plsc.VectorSubcoreMesh(core_axis_name="c", subcore_axis_name="s")


@functools.partial(
    pl.kernel,
    mesh=_mesh,
    compiler_params=pltpu.CompilerParams(use_tc_tiling_on_sc=True,
                                         needs_layout_passes=False),
    out_type=jax.ShapeDtypeStruct((T, D, B), jnp.float32),
    scratch_types=[
        pltpu.VMEM((8, 128), jnp.int32),
        pltpu.VMEM((1, 128), jnp.int32),
        pltpu.VMEM((128, 128), jnp.float32),
        pltpu.VMEM((D, 128), jnp.float32),
        pltpu.VMEM((T, D), jnp.float32),
        pltpu.SemaphoreType.DMA,
    ],
)
def _emb_kernel(xT_hbm, w2_hbm, pos_hbm, out_hbm,
                idxraw_v, row_v, g_v, o_v, pos_v, sem):
    wid = lax.axis_index("s") * _NC + lax.axis_index("c")
    b0 = wid * _BW

    pltpu.sync_copy(pos_hbm.at[pl.ds(0, T)], pos_v)
    lanes = lax.iota(jnp.int32, 16)

    def body_t(t, carry):
        r = lax.rem(t, 8)
        t8 = lax.div(t, 8)

        @pl.when(r == 0)
        def _load_idx_tile():
            pltpu.sync_copy(xT_hbm.at[pl.ds(t8 * 8, 8), pl.ds(b0, _BW)],
                            idxraw_v)

        h64s = []
        for g in range(8):
            raw = idxraw_v[r, pl.ds(16 * g, 16)]
            row_v[0, pl.ds(16 * g, 16)] = lax.shift_right_logical(raw, 1)
            h64s.append(lax.shift_left(raw & 1, 6))

        pltpu.async_copy(w2_hbm.at[row_v.at[0]], g_v, sem).wait()

        rowgs = [16 * g + lanes for g in range(8)]
        for c in range(4):
            pvec = pos_v[t, pl.ds(16 * c, 16)]
            for dl in range(16):
                d = 16 * c + dl
                p = pvec[dl]
                for g in range(8):
                    w = plsc.load_gather(g_v, [rowgs[g], h64s[g] + d])
                    o_v[d, pl.ds(16 * g, 16)] = w * SCALE + p

        pltpu.sync_copy(o_v, out_hbm.at[t, :, pl.ds(b0, _BW)])
        return carry

    lax.fori_loop(0, T, body_t, 0)


def kernel(x, W_word, W_pos):
    xT = x.T.astype(jnp.int32)
    W2 = W_word.reshape(VOCAB // 2, 2 * D)
    out3 = _emb_kernel(xT, W2, W_pos)
    return jnp.transpose(out3, (2, 0, 1))


# speedup vs baseline: 1.7601x; 1.7601x over previous
import functools

import jax
import jax.numpy as jnp
from jax import lax
from jax.experimental import pallas as pl
from jax.experimental.pallas import tpu as pltpu
from jax.experimental.pallas import tpu_sc as plsc

B = 4096
T = 200
D = 64
VOCAB = 1000000
SCALE = 8.0

_NC = 2
_NS = 16
_NW = _NC * _NS
_BW = B // _NW

_mesh = plsc.VectorSubcoreMesh(core_axis_name="c", subcore_axis_name="s")


@functools.partial(
    pl.kernel,
    mesh=_mesh,
    compiler_params=pltpu.CompilerParams(use_tc_tiling_on_sc=True,
                                         needs_layout_passes=False),
    out_type=jax.ShapeDtypeStruct((T, D, B), jnp.float32),
    scratch_types=[
        pltpu.VMEM((2, 8, 128), jnp.int32),
        pltpu.VMEM((2, 128), jnp.int32),
        pltpu.VMEM((2, 128), jnp.int32),
        pltpu.VMEM((2, 128, 128), jnp.float32),
        pltpu.VMEM((2, D, 144), jnp.float32),
        pltpu.VMEM((T, D), jnp.float32),
        pltpu.SemaphoreType.DMA((2,)),
        pltpu.SemaphoreType.DMA((2,)),
        pltpu.SemaphoreType.DMA((2,)),
    ],
)
def _emb_kernel(xT_hbm, w2_hbm, pos_hbm, out_hbm,
                idxraw_v, row_v, h64_v, g_v, o_v, pos_v,
                sem_i, sem_g, sem_o):
    wid = lax.axis_index("s") * _NC + lax.axis_index("c")
    b0 = wid * _BW

    pltpu.sync_copy(pos_hbm.at[pl.ds(0, T)], pos_v)
    lanes = lax.iota(jnp.int32, 16)

    def stage_indices(tile_slot, rr, chunk_slot):
        for g in range(8):
            raw = idxraw_v[tile_slot, rr, pl.ds(16 * g, 16)]
            row_v[chunk_slot, pl.ds(16 * g, 16)] = \
                lax.shift_right_logical(raw, 1)
            h64_v[chunk_slot, pl.ds(16 * g, 16)] = lax.shift_left(raw & 1, 6)

    def fire_gather(chunk_slot):
        pltpu.async_copy(w2_hbm.at[row_v.at[chunk_slot]],
                         g_v.at[chunk_slot], sem_g.at[chunk_slot])

    pltpu.sync_copy(xT_hbm.at[pl.ds(0, 8), pl.ds(b0, _BW)], idxraw_v.at[0])
    stage_indices(0, 0, 0)
    fire_gather(0)

    def body_t(t, carry):
        r = lax.rem(t, 8)
        t8 = lax.div(t, 8)
        slot = lax.rem(t, 2)
        nxt = t + 1
        nxt_slot = lax.rem(nxt, 2)
        nxt_tile = lax.div(nxt, 8)
        nxt_tile_slot = lax.rem(nxt_tile, 2)

        @pl.when((r == 0) & (t < 8 * (T // 8 - 1)))
        def _prefetch_idx_tile():
            nt = t8 + 1
            pltpu.async_copy(
                xT_hbm.at[pl.ds(nt * 8, 8), pl.ds(b0, _BW)],
                idxraw_v.at[lax.rem(nt, 2)], sem_i.at[lax.rem(nt, 2)])

        @pl.when(nxt < T)
        def _stage_next():
            @pl.when(r == 7)
            def _wait_tile():
                pltpu.make_async_copy(
                    xT_hbm.at[pl.ds(0, 8), pl.ds(b0, _BW)],
                    idxraw_v.at[nxt_tile_slot],
                    sem_i.at[nxt_tile_slot]).wait()
            stage_indices(nxt_tile_slot, lax.rem(nxt, 8), nxt_slot)
            fire_gather(nxt_slot)

        pltpu.make_async_copy(w2_hbm.at[row_v.at[slot]], g_v.at[slot],
                              sem_g.at[slot]).wait()

        @pl.when(t >= 2)
        def _wait_out():
            pltpu.make_async_copy(o_v.at[slot, :, pl.ds(0, 128)],
                                  out_hbm.at[0, :, pl.ds(b0, _BW)],
                                  sem_o.at[slot]).wait()

        gcur = g_v.at[slot]
        h64cur = h64_v.at[slot]
        ocur = o_v.at[slot]
        pvecs = [pos_v[t, pl.ds(16 * c, 16)] for c in range(4)]
        didx = [16 * c + lanes for c in range(4)]

        @plsc.parallel_loop(0, _BW, 1, unroll=4)
        def _compute(tok):
            tok_splat = jnp.broadcast_to(tok, (16,))
            h64b = plsc.load_gather(h64cur, [tok_splat])
            for c in range(4):
                w = plsc.load_gather(gcur, [tok_splat, h64b + didx[c]])
                plsc.store_scatter(ocur, [didx[c], tok_splat],
                                   w * SCALE + pvecs[c])

        pltpu.async_copy(o_v.at[slot, :, pl.ds(0, 128)],
                         out_hbm.at[t, :, pl.ds(b0, _BW)],
                         sem_o.at[slot])
        return carry

    lax.fori_loop(0, T, body_t, 0)

    for s in range(2):
        pltpu.make_async_copy(o_v.at[s, :, pl.ds(0, 128)],
                              out_hbm.at[0, :, pl.ds(b0, _BW)],
                              sem_o.at[s]).wait()


def kernel(x, W_word, W_pos):
    xT = x.T.astype(jnp.int32)
    W2 = W_word.reshape(VOCAB // 2, 2 * D)
    out3 = _emb_kernel(xT, W2, W_pos)
    return jnp.transpose(out3, (2, 0, 1))


# speedup vs baseline: 1.9117x; 1.0861x over previous
import functools

import jax
import jax.numpy as jnp
from jax import lax
from jax.experimental import pallas as pl
from jax.experimental.pallas import tpu as pltpu
from jax.experimental.pallas import tpu_sc as plsc

B = 4096
T = 200
D = 64
VOCAB = 1000000
SCALE = 8.0

_NC = 2
_NS = 16
_NW = _NC * _NS
_BW = B // _NW
_NG = 3

_mesh = plsc.VectorSubcoreMesh(core_axis_name="c", subcore_axis_name="s")


@functools.partial(
    pl.kernel,
    mesh=_mesh,
    compiler_params=pltpu.CompilerParams(use_tc_tiling_on_sc=True,
                                         needs_layout_passes=False),
    out_type=jax.ShapeDtypeStruct((T, D, B), jnp.float32),
    scratch_types=[
        pltpu.VMEM((2, 8, 128), jnp.int32),
        pltpu.VMEM((_NG, 128), jnp.int32),
        pltpu.VMEM((_NG, 128), jnp.int32),
        pltpu.VMEM((_NG, 128, 128), jnp.float32),
        pltpu.VMEM((2, D, 128), jnp.float32),
        pltpu.VMEM((T, D), jnp.float32),
        pltpu.SemaphoreType.DMA((2,)),
        pltpu.SemaphoreType.DMA((_NG,)),
        pltpu.SemaphoreType.DMA((2,)),
    ],
)
def _emb_kernel(xT_hbm, w2_hbm, pos_hbm, out_hbm,
                idxraw_v, row_v, h64_v, g_v, o_v, pos_v,
                sem_i, sem_g, sem_o):
    wid = lax.axis_index("s") * _NC + lax.axis_index("c")
    b0 = wid * _BW

    pltpu.sync_copy(pos_hbm.at[pl.ds(0, T)], pos_v)
    lanes = lax.iota(jnp.int32, 16)
    zero16 = lanes - lanes

    def stage_indices(tile_slot, rr, chunk_slot):
        for g in range(8):
            raw = idxraw_v[tile_slot, rr, pl.ds(16 * g, 16)]
            row_v[chunk_slot, pl.ds(16 * g, 16)] = \
                lax.shift_right_logical(raw, 1)
            h64_v[chunk_slot, pl.ds(16 * g, 16)] = lax.shift_left(raw & 1, 6)

    def fire_gather(chunk_slot):
        pltpu.async_copy(w2_hbm.at[row_v.at[chunk_slot]],
                         g_v.at[chunk_slot], sem_g.at[chunk_slot])

    pltpu.sync_copy(xT_hbm.at[pl.ds(0, 8), pl.ds(b0, _BW)], idxraw_v.at[0])
    stage_indices(0, 0, 0)
    fire_gather(0)
    stage_indices(0, 1, 1)
    fire_gather(1)

    def body_t(t, carry):
        r = lax.rem(t, 8)
        t8 = lax.div(t, 8)
        slot = lax.rem(t, _NG)
        oslot = lax.rem(t, 2)
        nxt2 = t + 2
        nxt2_slot = lax.rem(nxt2, _NG)
        nxt2_tile_slot = lax.rem(lax.div(nxt2, 8), 2)

        @pl.when((r == 0) & (t < 8 * (T // 8 - 1)))
        def _prefetch_idx_tile():
            nt = t8 + 1
            pltpu.async_copy(
                xT_hbm.at[pl.ds(nt * 8, 8), pl.ds(b0, _BW)],
                idxraw_v.at[lax.rem(nt, 2)], sem_i.at[lax.rem(nt, 2)])

        @pl.when(nxt2 < T)
        def _stage_next():
            @pl.when(r == 6)
            def _wait_tile():
                pltpu.make_async_copy(
                    xT_hbm.at[pl.ds(0, 8), pl.ds(b0, _BW)],
                    idxraw_v.at[nxt2_tile_slot],
                    sem_i.at[nxt2_tile_slot]).wait()
            stage_indices(nxt2_tile_slot, lax.rem(nxt2, 8), nxt2_slot)
            fire_gather(nxt2_slot)

        pltpu.make_async_copy(w2_hbm.at[row_v.at[slot]], g_v.at[slot],
                              sem_g.at[slot]).wait()

        @pl.when(t >= 2)
        def _wait_out():
            pltpu.make_async_copy(o_v.at[oslot],
                                  out_hbm.at[0, :, pl.ds(b0, _BW)],
                                  sem_o.at[oslot]).wait()

        gcur = g_v.at[slot]
        bases = [h64_v[slot, pl.ds(16 * g, 16)] +
                 lax.shift_left(16 * g + lanes, 7) for g in range(8)]
        tsplat = jnp.broadcast_to(t, (16,)).astype(jnp.int32)

        @plsc.parallel_loop(0, D, 1, unroll=16)
        def _compute(d):
            dsplat = jnp.broadcast_to(d, (16,))
            p = plsc.load_gather(pos_v, [tsplat, dsplat])
            for g in range(8):
                w = plsc.load_gather(gcur, [zero16, bases[g] + d])
                o_v[oslot, d, pl.ds(16 * g, 16)] = w * SCALE + p

        pltpu.async_copy(o_v.at[oslot], out_hbm.at[t, :, pl.ds(b0, _BW)],
                         sem_o.at[oslot])
        return carry

    lax.fori_loop(0, T, body_t, 0)

    for s in range(2):
        pltpu.make_async_copy(o_v.at[s], out_hbm.at[0, :, pl.ds(b0, _BW)],
                              sem_o.at[s]).wait()


def kernel(x, W_word, W_pos):
    xT = x.T.astype(jnp.int32)
    W2 = W_word.reshape(VOCAB // 2, 2 * D)
    out3 = _emb_kernel(xT, W2, W_pos)
    return jnp.transpose(out3, (2, 0, 1))
